# baseline (speedup 1.0000x reference)
import functools

import jax
import jax.numpy as jnp
from jax import lax
from jax.experimental import pallas as pl
from jax.experimental.pallas import tpu as pltpu
from jax.experimental.pallas import tpu_sc as plsc

N = 50000
E = 800000
D = 64
T = 4
WID = 128
NC = 2
NS = 16
CH = 2304
P = 11
ROWS = CH * T
RPT = ROWS // NS
GRP = 64
NB = 3
ACC_ROWS = ROWS + GRP
DUMMY = ROWS
OUT_ROWS = NC * CH * P * T
EPT = E // NS
CE = 2000
RCH = 5
NR = EPT // (CE * RCH)
CAP = CE * RCH + 176
ZR = 32


def _sc_kernel(xp_h, src_h, key_h, out_h,
               pk_list, es_a, ek_a, es_b, ek_b,
               src_0, idx_0, src_1, idx_1, src_2, idx_2,
               rows_0, rows_1, rows_2, zbuf, acc,
               esem_a, esem_b, gsem_0, gsem_1, gsem_2,
               ssem_0, ssem_1, ssem_2, zsem):
    c = lax.axis_index("c")
    s = lax.axis_index("s")

    zero16 = jnp.zeros((16,), jnp.float32)

    def zb_body(r, _):
        for cg in range(WID // 16):
            zbuf[r, pl.ds(cg * 16, 16)] = zero16
        return 0

    lax.fori_loop(0, ZR, zb_body, 0)

    iota16 = lax.iota(jnp.int32, 16)
    dummy = jnp.full((16,), DUMMY, jnp.int32)
    ebufs = [(es_a, ek_a), (es_b, ek_b)]
    esems = [esem_a, esem_b]

    def issue_edges(base, sl):
        hs = pltpu.async_copy(src_h.at[pl.ds(base, CE)], ebufs[sl][0],
                              esems[sl])
        hk = pltpu.async_copy(key_h.at[pl.ds(base, CE)], ebufs[sl][1],
                              esems[sl])
        return (hs, hk)

    srcs = [src_0, src_1, src_2]
    idxs = [idx_0, idx_1, idx_2]
    rows = [rows_0, rows_1, rows_2]
    gsems = [gsem_0, gsem_1, gsem_2]
    ssems = [ssem_0, ssem_1, ssem_2]

    def unpack(g, srcst, idxst):
        for j in range(GRP // 16):
            pk = pk_list[pl.ds(g * GRP + j * 16, 16)]
            srcst[pl.ds(j * 16, 16)] = pk >> 15
            idxst[pl.ds(j * 16, 16)] = pk & 32767

    def issue_gather(b):
        pltpu.async_copy(xp_h.at[srcs[b]], rows[b], gsems[b])

    def wait_gather(b):
        pltpu.make_async_copy(xp_h.at[srcs[b]], rows[b], gsems[b]).wait()

    def issue_scatter(b):
        pltpu.async_copy(rows[b], acc.at[idxs[b]], ssems[b], add=True)

    def wait_scatter(b):
        pltpu.make_async_copy(rows[b], acc.at[idxs[b]], ssems[b]).wait()

    def pass_body(p, _):
        lo4 = ((p * NC + c) * CH) * T

        zbase = pl.multiple_of(s * RPT, 8)
        def zinit_issue(k, _):
            pltpu.async_copy(zbuf, acc.at[pl.ds(zbase + k * ZR, ZR)], zsem)
            return 0

        lax.fori_loop(0, RPT // ZR, zinit_issue, 0)

        def zinit_drain(k, _):
            pltpu.make_async_copy(zbuf, acc.at[pl.ds(zbase, ZR)], zsem).wait()
            return 0

        lax.fori_loop(0, RPT // ZR, zinit_drain, 0)
        plsc.subcore_barrier()

        def round_body(r, _):
            base0 = pl.multiple_of(s * EPT + r * (RCH * CE), 8)
            h_e = [None, None]
            h_e[0] = issue_edges(base0, 0)

            pos = 0
            for ch in range(RCH):
                sl = ch % 2
                h_e[sl][0].wait()
                h_e[sl][1].wait()
                if ch + 1 < RCH:
                    nb = (ch + 1) % 2
                    nbase = pl.multiple_of(base0 + (ch + 1) * CE, 8)
                    h_e[nb] = issue_edges(nbase, nb)
                es, ek = ebufs[sl]

                def grp_body(g, pos, es=es, ek=ek):
                    sv = es[pl.ds(g * 16, 16)]
                    kv = ek[pl.ds(g * 16, 16)]
                    m = (kv >= lo4) & (kv < lo4 + CH * T)
                    li = kv - lo4
                    packed = (sv << 15) | li
                    mi = m.astype(jnp.int32)
                    csum = plsc.cumsum(mi)
                    plsc.store_scatter(pk_list, [pos + csum - mi], packed,
                                       mask=m)
                    return pos + jnp.sum(mi)

                pos = lax.fori_loop(0, CE // 16, grp_body, pos)

            for j in range(GRP // 16):
                plsc.store_scatter(pk_list, [pos + j * 16 + iota16], dummy)

            ngr = (pos + GRP - 1) // GRP

            for b in range(NB):
                @pl.when(b < ngr)
                def _(b=b):
                    unpack(b, srcs[b], idxs[b])
                    issue_gather(b)

            def proc(h, _):
                g0 = NB * h
                for b in range(NB):
                    @pl.when(g0 + b < ngr)
                    def _(b=b):
                        wait_gather(b)
                        issue_scatter(b)
                for b in range(NB):
                    @pl.when(g0 + b + NB < ngr)
                    def _(b=b, g=g0):
                        wait_scatter(b)
                        unpack(g + b + NB, srcs[b], idxs[b])
                        issue_gather(b)
                return 0

            lax.fori_loop(0, (ngr + NB - 1) // NB, proc, 0)

            for b in range(NB):
                @pl.when(b < ngr)
                def _(b=b):
                    wait_scatter(b)
            return 0

        lax.fori_loop(0, NR, round_body, 0)
        plsc.subcore_barrier()

        pltpu.sync_copy(acc.at[pl.ds(zbase, RPT)],
                        out_h.at[pl.ds(pl.multiple_of(lo4 + s * RPT, 8), RPT)])
        return 0

    lax.fori_loop(0, P, pass_body, 0)


def _make_sc():
    mesh = plsc.VectorSubcoreMesh(core_axis_name="c", subcore_axis_name="s")
    return pl.kernel(
        _sc_kernel,
        mesh=mesh,
        compiler_params=pltpu.CompilerParams(needs_layout_passes=False),
        out_type=jax.ShapeDtypeStruct((OUT_ROWS, WID), jnp.float32),
        scratch_types=[
            pltpu.VMEM((CAP,), jnp.int32),
            pltpu.VMEM((CE,), jnp.int32),
            pltpu.VMEM((CE,), jnp.int32),
            pltpu.VMEM((CE,), jnp.int32),
            pltpu.VMEM((CE,), jnp.int32),
            pltpu.VMEM((GRP,), jnp.int32),
            pltpu.VMEM((GRP,), jnp.int32),
            pltpu.VMEM((GRP,), jnp.int32),
            pltpu.VMEM((GRP,), jnp.int32),
            pltpu.VMEM((GRP,), jnp.int32),
            pltpu.VMEM((GRP,), jnp.int32),
            pltpu.VMEM((GRP, WID), jnp.float32),
            pltpu.VMEM((GRP, WID), jnp.float32),
            pltpu.VMEM((GRP, WID), jnp.float32),
            pltpu.VMEM((ZR, WID), jnp.float32),
            pltpu.VMEM_SHARED((ACC_ROWS, WID), jnp.float32),
            pltpu.SemaphoreType.DMA,
            pltpu.SemaphoreType.DMA,
            pltpu.SemaphoreType.DMA,
            pltpu.SemaphoreType.DMA,
            pltpu.SemaphoreType.DMA,
            pltpu.SemaphoreType.DMA,
            pltpu.SemaphoreType.DMA,
            pltpu.SemaphoreType.DMA,
            pltpu.SemaphoreType.DMA,
        ],
    )


BN = 2000


def _dense_kernel(s_ref, w_ref, b_ref, o_ref, *, pad_out):
    S = s_ref[...]
    parts = []
    for t in range(T):
        st = S[:, t * WID:t * WID + D]
        cnt = S[:, t * WID + D:t * WID + D + 1]
        mean = st / jnp.maximum(cnt, 1.0)
        nr = 1.0 - jnp.sum(mean * mean, axis=1, keepdims=True)
        parts.append(mean / nr)
    h = jnp.concatenate(parts, axis=1)
    y = jnp.dot(h, w_ref[...].T, preferred_element_type=jnp.float32)
    y = jnp.maximum(y + b_ref[...], 0.0)
    if pad_out:
        o_ref[:, :D] = y
        o_ref[:, D:D + 1] = jnp.ones((BN, 1), jnp.float32)
        o_ref[:, D + 1:] = jnp.zeros((BN, WID - D - 1), jnp.float32)
    else:
        o_ref[...] = y


def _dense(sums, w, b, pad_out):
    ow = WID if pad_out else D
    return pl.pallas_call(
        functools.partial(_dense_kernel, pad_out=pad_out),
        grid=(N // BN,),
        in_specs=[
            pl.BlockSpec((BN, T * WID), lambda i: (i, 0)),
            pl.BlockSpec((D, T * D), lambda i: (0, 0)),
            pl.BlockSpec((1, D), lambda i: (0, 0)),
        ],
        out_specs=pl.BlockSpec((BN, ow), lambda i: (i, 0)),
        out_shape=jax.ShapeDtypeStruct((N, ow), jnp.float32),
    )(sums, w, b.reshape(1, D))


def kernel(x, edge_index, time_index, W1, b1, W2, b2):
    src = edge_index[0]
    dst = edge_index[1]
    key = dst * T + time_index.astype(jnp.int32)
    xp = jnp.concatenate(
        [x, jnp.ones((N, 1), jnp.float32), jnp.zeros((N, WID - D - 1), jnp.float32)],
        axis=1)
    sc = _make_sc()
    s1 = sc(xp, src, key)
    h = _dense(s1.reshape(-1, T * WID), W1, b1, pad_out=True)
    s2 = sc(h, src, key)
    out = _dense(s2.reshape(-1, T * WID), W2, b2, pad_out=False)
    return out


# speedup vs baseline: 1.1334x; 1.1334x over previous
import functools

import jax
import jax.numpy as jnp
from jax import lax
from jax.experimental import pallas as pl
from jax.experimental.pallas import tpu as pltpu
from jax.experimental.pallas import tpu_sc as plsc

N = 50000
E = 800000
D = 64
T = 4
WID = 128
NC = 2
NS = 16
CH = 2560
P = 10
ROWS = CH * T
RPT = ROWS // NS
GRP = 64
ACC_ROWS = ROWS + GRP
DUMMY = ROWS
OUT_ROWS = NC * CH * P * T
EPT = E // NS
CE = 2000
RCH = 5
NR = EPT // (CE * RCH)
CAP = CE * RCH + 176
ZR = 32


def _sc_kernel(xp_h, src_h, key_h, out_h,
               pk_list, es_a, ek_a, es_b, ek_b,
               src_a0, src_a1, idx_a, src_b0, src_b1, idx_b,
               rows_a, rows_b, zbuf, acc,
               esem_a, esem_b, gsem_a, gsem_b, zsem):
    c = lax.axis_index("c")
    s = lax.axis_index("s")

    zero16 = jnp.zeros((16,), jnp.float32)

    def zb_body(r, _):
        for cg in range(WID // 16):
            zbuf[r, pl.ds(cg * 16, 16)] = zero16
        return 0

    lax.fori_loop(0, ZR, zb_body, 0)

    iota16 = lax.iota(jnp.int32, 16)
    dummy = jnp.full((16,), DUMMY, jnp.int32)
    ebufs = [(es_a, ek_a), (es_b, ek_b)]
    esems = [esem_a, esem_b]

    def issue_edges(base, sl):
        hs = pltpu.async_copy(src_h.at[pl.ds(base, CE)], ebufs[sl][0],
                              esems[sl])
        hk = pltpu.async_copy(key_h.at[pl.ds(base, CE)], ebufs[sl][1],
                              esems[sl])
        return (hs, hk)

    HG = GRP // 2

    def unpack(g, src0, src1, idxst):
        for j in range(GRP // 16):
            pk = pk_list[pl.ds(g * GRP + j * 16, 16)]
            half = (src0, src1)[j // (HG // 16)]
            half[pl.ds((j % (HG // 16)) * 16, 16)] = pk >> 15
            idxst[pl.ds(j * 16, 16)] = pk & 32767

    def issue_gather(src0, src1, rowsb, gsem):
        pltpu.async_copy(xp_h.at[src0], rowsb.at[pl.ds(0, HG)], gsem)
        pltpu.async_copy(xp_h.at[src1], rowsb.at[pl.ds(HG, HG)], gsem)

    def wait_gather(src0, src1, rowsb, gsem):
        pltpu.make_async_copy(xp_h.at[src0], rowsb.at[pl.ds(0, HG)],
                              gsem).wait()
        pltpu.make_async_copy(xp_h.at[src1], rowsb.at[pl.ds(HG, HG)],
                              gsem).wait()

    def pass_body(p, _):
        lo4 = ((p * NC + c) * CH) * T

        zbase = pl.multiple_of(s * RPT, 8)
        def zinit_issue(k, _):
            pltpu.async_copy(zbuf, acc.at[pl.ds(zbase + k * ZR, ZR)], zsem)
            return 0

        lax.fori_loop(0, RPT // ZR, zinit_issue, 0)

        def zinit_drain(k, _):
            pltpu.make_async_copy(zbuf, acc.at[pl.ds(zbase, ZR)], zsem).wait()
            return 0

        lax.fori_loop(0, RPT // ZR, zinit_drain, 0)
        plsc.subcore_barrier()

        def round_body(r, _):
            base0 = pl.multiple_of(s * EPT + r * (RCH * CE), 8)
            h_e = [None, None]
            h_e[0] = issue_edges(base0, 0)

            pos = 0
            for ch in range(RCH):
                sl = ch % 2
                h_e[sl][0].wait()
                h_e[sl][1].wait()
                if ch + 1 < RCH:
                    nb = (ch + 1) % 2
                    nbase = pl.multiple_of(base0 + (ch + 1) * CE, 8)
                    h_e[nb] = issue_edges(nbase, nb)
                es, ek = ebufs[sl]

                def grp_body(g, pos, es=es, ek=ek):
                    sv = es[pl.ds(g * 16, 16)]
                    kv = ek[pl.ds(g * 16, 16)]
                    m = (kv >= lo4) & (kv < lo4 + CH * T)
                    li = kv - lo4
                    packed = (sv << 15) | li
                    mi = m.astype(jnp.int32)
                    csum = plsc.cumsum(mi)
                    plsc.store_scatter(pk_list, [pos + csum - mi], packed,
                                       mask=m)
                    return pos + jnp.sum(mi)

                pos = lax.fori_loop(0, CE // 16, grp_body, pos)

            for j in range(GRP // 16):
                plsc.store_scatter(pk_list, [pos + j * 16 + iota16], dummy)

            ngr = (pos + GRP - 1) // GRP

            @pl.when(ngr > 0)
            def _():
                unpack(0, src_a0, src_a1, idx_a)
                issue_gather(src_a0, src_a1, rows_a, gsem_a)

            def proc2(h, _):
                g1 = 2 * h + 1

                @pl.when(g1 < ngr)
                def _():
                    unpack(g1, src_b0, src_b1, idx_b)
                    issue_gather(src_b0, src_b1, rows_b, gsem_b)

                wait_gather(src_a0, src_a1, rows_a, gsem_a)
                pltpu.sync_copy(rows_a, acc.at[idx_a], add=True)

                @pl.when(2 * h + 2 < ngr)
                def _():
                    unpack(2 * h + 2, src_a0, src_a1, idx_a)
                    issue_gather(src_a0, src_a1, rows_a, gsem_a)

                @pl.when(g1 < ngr)
                def _():
                    wait_gather(src_b0, src_b1, rows_b, gsem_b)
                    pltpu.sync_copy(rows_b, acc.at[idx_b], add=True)

                return 0

            lax.fori_loop(0, (ngr + 1) // 2, proc2, 0)
            return 0

        lax.fori_loop(0, NR, round_body, 0)
        plsc.subcore_barrier()

        pltpu.sync_copy(acc.at[pl.ds(zbase, RPT)],
                        out_h.at[pl.ds(pl.multiple_of(lo4 + s * RPT, 8), RPT)])
        return 0

    lax.fori_loop(0, P, pass_body, 0)


def _make_sc():
    mesh = plsc.VectorSubcoreMesh(core_axis_name="c", subcore_axis_name="s")
    return pl.kernel(
        _sc_kernel,
        mesh=mesh,
        compiler_params=pltpu.CompilerParams(needs_layout_passes=False),
        out_type=jax.ShapeDtypeStruct((OUT_ROWS, WID), jnp.float32),
        scratch_types=[
            pltpu.VMEM((CAP,), jnp.int32),
            pltpu.VMEM((CE,), jnp.int32),
            pltpu.VMEM((CE,), jnp.int32),
            pltpu.VMEM((CE,), jnp.int32),
            pltpu.VMEM((CE,), jnp.int32),
            pltpu.VMEM((GRP // 2,), jnp.int32),
            pltpu.VMEM((GRP // 2,), jnp.int32),
            pltpu.VMEM((GRP,), jnp.int32),
            pltpu.VMEM((GRP // 2,), jnp.int32),
            pltpu.VMEM((GRP // 2,), jnp.int32),
            pltpu.VMEM((GRP,), jnp.int32),
            pltpu.VMEM((GRP, WID), jnp.float32),
            pltpu.VMEM((GRP, WID), jnp.float32),
            pltpu.VMEM((ZR, WID), jnp.float32),
            pltpu.VMEM_SHARED((ACC_ROWS, WID), jnp.float32),
            pltpu.SemaphoreType.DMA,
            pltpu.SemaphoreType.DMA,
            pltpu.SemaphoreType.DMA,
            pltpu.SemaphoreType.DMA,
            pltpu.SemaphoreType.DMA,
        ],
    )


BN = 2000


def _dense_kernel(s_ref, w_ref, b_ref, o_ref, *, pad_out):
    S = s_ref[...]
    parts = []
    for t in range(T):
        st = S[:, t * WID:t * WID + D]
        cnt = S[:, t * WID + D:t * WID + D + 1]
        mean = st / jnp.maximum(cnt, 1.0)
        nr = 1.0 - jnp.sum(mean * mean, axis=1, keepdims=True)
        parts.append(mean / nr)
    h = jnp.concatenate(parts, axis=1)
    y = jnp.dot(h, w_ref[...].T, preferred_element_type=jnp.float32)
    y = jnp.maximum(y + b_ref[...], 0.0)
    if pad_out:
        o_ref[:, :D] = y
        o_ref[:, D:D + 1] = jnp.ones((BN, 1), jnp.float32)
        o_ref[:, D + 1:] = jnp.zeros((BN, WID - D - 1), jnp.float32)
    else:
        o_ref[...] = y


def _dense(sums, w, b, pad_out):
    ow = WID if pad_out else D
    return pl.pallas_call(
        functools.partial(_dense_kernel, pad_out=pad_out),
        grid=(N // BN,),
        in_specs=[
            pl.BlockSpec((BN, T * WID), lambda i: (i, 0)),
            pl.BlockSpec((D, T * D), lambda i: (0, 0)),
            pl.BlockSpec((1, D), lambda i: (0, 0)),
        ],
        out_specs=pl.BlockSpec((BN, ow), lambda i: (i, 0)),
        out_shape=jax.ShapeDtypeStruct((N, ow), jnp.float32),
    )(sums, w, b.reshape(1, D))


def kernel(x, edge_index, time_index, W1, b1, W2, b2):
    src = edge_index[0]
    dst = edge_index[1]
    key = dst * T + time_index.astype(jnp.int32)
    xp = jnp.concatenate(
        [x, jnp.ones((N, 1), jnp.float32), jnp.zeros((N, WID - D - 1), jnp.float32)],
        axis=1)
    sc = _make_sc()
    s1 = sc(xp, src, key)
    h = _dense(s1.reshape(-1, T * WID), W1, b1, pad_out=True)
    s2 = sc(h, src, key)
    out = _dense(s2.reshape(-1, T * WID), W2, b2, pad_out=False)
    return out
